# baseline (speedup 1.0000x reference)
import functools

import numpy as np
import jax
import jax.numpy as jnp
from jax import lax
from jax.experimental import pallas as pl
from jax.experimental.pallas import tpu as pltpu
from jax.experimental.pallas import tpu_sc as plsc

_C = 192
_K = 16
_P = 3
_CLAMP = 1.5
_NSPAN = _K - _P
_SPAD = 16
_INVH = 6.5


def _knots_f64():
    n_int = _K - _P - 1
    interior = np.linspace(-1.0, 1.0, n_int + 2)[1:-1]
    return np.concatenate(
        [np.full(_P + 1, -1.0), interior, np.full(_P + 1, 1.0)])


def _local_basis_f64(x, i, kn):
    js = np.arange(1, _P + 1)
    left = x - kn[i + 1 - js]
    right = kn[i + js] - x
    N = np.zeros(_P + 1)
    N[0] = 1.0
    for j in range(1, _P + 1):
        saved = 0.0
        for r in range(j):
            denom = right[r] + left[j - r - 1]
            temp = N[r] / denom
            N[r] = saved + right[r] * temp
            saved = left[j - r - 1] * temp
        N[j] = saved
    return N


def _basis_matrices():
    kn = _knots_f64()
    M = np.zeros((_K, 4 * _SPAD))
    for s in range(_NSPAN):
        i = s + _P
        t0, t1 = kn[i], kn[i + 1]
        xs = t0 + (t1 - t0) * np.array([0.1, 0.35, 0.65, 0.9])
        V = np.vander(xs, 4, increasing=True)
        Nm = np.stack([_local_basis_f64(x, i, kn) for x in xs])
        Bj = np.linalg.solve(V, Nm).T
        for j in range(4):
            for d in range(4):
                M[s + j, d * _SPAD + s] = Bj[j, d]
    mask = np.zeros((1, 4 * _SPAD))
    mask[0, :_NSPAN] = 1.0
    return M.astype(np.float32), mask.astype(np.float32)


_M_NP, _MSK_NP = _basis_matrices()


def _prep_body(alpha_ref, scal_ref, mat_ref, msk_ref, coef_ref, srec_ref):
    alpha = alpha_ref[:]
    coef = jnp.dot(alpha, mat_ref[:], preferred_element_type=jnp.float32,
                   precision=lax.Precision.HIGHEST)
    coef_ref[:] = coef + scal_ref[:, 3:4] * msk_ref[:]
    srec_ref[:] = jnp.concatenate(
        [scal_ref[:, 0:3], jnp.zeros((_C, 5), jnp.float32)], axis=1)


def _prep_tables(alpha, scal):
    return pl.pallas_call(
        _prep_body,
        out_shape=[
            jax.ShapeDtypeStruct((_C, 4 * _SPAD), jnp.float32),
            jax.ShapeDtypeStruct((_C, 8), jnp.float32),
        ],
    )(alpha, scal, jnp.asarray(_M_NP), jnp.asarray(_MSK_NP))


def _make_sc_kernel(total):
    slice_elems = 224 * 224
    cps = 4
    chunk = slice_elems // cps
    nw = 32
    chunks = total // chunk
    assert chunks % nw == 0
    cpw = chunks // nw
    assert cpw % 2 == 0
    vregs = chunk // 16
    plane = _C * _SPAD

    mesh = plsc.VectorSubcoreMesh(core_axis_name="c", subcore_axis_name="s")

    @functools.partial(
        pl.kernel,
        out_type=jax.ShapeDtypeStruct((total,), jnp.float32),
        mesh=mesh,
        scratch_types=[
            [pltpu.VMEM((plane,), jnp.float32) for _ in range(4)],
            pltpu.VMEM((_C * 8,), jnp.float32),
            [pltpu.VMEM((chunk,), jnp.float32) for _ in range(2)],
            [pltpu.VMEM((chunk,), jnp.float32) for _ in range(2)],
            [pltpu.SemaphoreType.DMA for _ in range(4)],
        ],
        compiler_params=pltpu.CompilerParams(needs_layout_passes=False),
    )
    def sc_kernel(x_hbm, planes_hbm, srec_hbm, out_hbm,
                  tbl_v, srec_v, xin_v, yout_v, sems):
        wid = lax.axis_index("s") * 2 + lax.axis_index("c")
        for d in range(4):
            pltpu.sync_copy(planes_hbm.at[pl.ds(d * plane, plane)], tbl_v[d])
        pltpu.sync_copy(srec_hbm, srec_v)
        g0 = wid * cpw

        def in_copy(k, buf, sem):
            return pltpu.make_async_copy(
                x_hbm.at[pl.ds((g0 + k) * chunk, chunk)], buf, sem)

        def out_copy(k, buf, sem):
            return pltpu.make_async_copy(
                buf, out_hbm.at[pl.ds((g0 + k) * chunk, chunk)], sem)

        def compute(k, xv, yv):
            c16 = (((g0 + k) // cps) % _C) * _SPAD
            c16f = c16.astype(jnp.float32)
            ccv = jnp.full((16,), c16f + _INVH)
            blov = jnp.full((16,), c16f)
            bhiv = jnp.full((16,), c16f + 12.0)
            sidx = jnp.full((16,), (((g0 + k) // cps) % _C) * 8, jnp.int32)
            av = plsc.load_gather(srec_v, [sidx])
            bv = plsc.load_gather(srec_v, [sidx + 1])
            gv = plsc.load_gather(srec_v, [sidx + 2])

            @plsc.parallel_loop(0, vregs, unroll=8)
            def _vloop(i):
                xx = xv[pl.ds(i * 16, 16)]
                xa = jnp.minimum(jnp.maximum(xx * av + bv, -_CLAMP), _CLAMP)
                t = jnp.minimum(jnp.maximum(xa * _INVH + ccv, blov), bhiv)
                s = t.astype(jnp.int32)
                c0 = plsc.load_gather(tbl_v[0], [s])
                c1 = plsc.load_gather(tbl_v[1], [s])
                c2 = plsc.load_gather(tbl_v[2], [s])
                c3 = plsc.load_gather(tbl_v[3], [s])
                r = ((c3 * xa + c2) * xa + c1) * xa + c0
                yv[pl.ds(i * 16, 16)] = xx * gv + r

        in_copy(0, xin_v[0], sems[0]).start()

        @pl.loop(0, cpw // 2)
        def _outer(k2):
            k = k2 * 2
            in_copy(k + 1, xin_v[1], sems[1]).start()
            in_copy(k, xin_v[0], sems[0]).wait()

            @pl.when(k2 > 0)
            def _():
                out_copy(k - 2, yout_v[0], sems[2]).wait()

            compute(k, xin_v[0], yout_v[0])
            out_copy(k, yout_v[0], sems[2]).start()

            @pl.when(k + 2 < cpw)
            def _():
                in_copy(k + 2, xin_v[0], sems[0]).start()

            in_copy(k + 1, xin_v[1], sems[1]).wait()

            @pl.when(k2 > 0)
            def _():
                out_copy(k - 1, yout_v[1], sems[3]).wait()

            compute(k + 1, xin_v[1], yout_v[1])
            out_copy(k + 1, yout_v[1], sems[3]).start()

        out_copy(cpw - 2, yout_v[0], sems[2]).wait()
        out_copy(cpw - 1, yout_v[1], sems[3]).wait()

    return sc_kernel


def kernel(x, a, b, alpha, id_gain, bias):
    scal = jnp.stack([a, b, id_gain, bias], axis=1)
    coef, srec = _prep_tables(alpha, scal)
    planes = coef.reshape(_C, 4, _SPAD).transpose(1, 0, 2).reshape(-1)
    total = x.size
    sc = _make_sc_kernel(total)
    y = sc(x.reshape(total), planes, srec.reshape(-1))
    return y.reshape(x.shape)


# speedup vs baseline: 1.5407x; 1.5407x over previous
import functools

import numpy as np
import jax
import jax.numpy as jnp
from jax import lax
from jax.experimental import pallas as pl
from jax.experimental.pallas import tpu as pltpu
from jax.experimental.pallas import tpu_sc as plsc

_C = 192
_K = 16
_P = 3
_CLAMP = 1.5
_NSPAN = _K - _P
_SPAD = 16
_INVH = 6.5


def _knots_f64():
    n_int = _K - _P - 1
    interior = np.linspace(-1.0, 1.0, n_int + 2)[1:-1]
    return np.concatenate(
        [np.full(_P + 1, -1.0), interior, np.full(_P + 1, 1.0)])


def _local_basis_f64(x, i, kn):
    js = np.arange(1, _P + 1)
    left = x - kn[i + 1 - js]
    right = kn[i + js] - x
    N = np.zeros(_P + 1)
    N[0] = 1.0
    for j in range(1, _P + 1):
        saved = 0.0
        for r in range(j):
            denom = right[r] + left[j - r - 1]
            temp = N[r] / denom
            N[r] = saved + right[r] * temp
            saved = left[j - r - 1] * temp
        N[j] = saved
    return N


def _basis_matrices():
    kn = _knots_f64()
    M = np.zeros((_K, 4 * _SPAD))
    for s in range(_NSPAN):
        i = s + _P
        t0, t1 = kn[i], kn[i + 1]
        xs = t0 + (t1 - t0) * np.array([0.1, 0.35, 0.65, 0.9])
        V = np.vander(xs, 4, increasing=True)
        Nm = np.stack([_local_basis_f64(x, i, kn) for x in xs])
        Bj = np.linalg.solve(V, Nm).T
        for j in range(4):
            for d in range(4):
                M[s + j, d * _SPAD + s] = Bj[j, d]
    mask = np.zeros((1, 4 * _SPAD))
    mask[0, :_NSPAN] = 1.0
    return M.astype(np.float32), mask.astype(np.float32)


_M_NP, _MSK_NP = _basis_matrices()


def _prep_body(alpha_ref, scal_ref, mat_ref, msk_ref, coef_ref, srec_ref):
    alpha = alpha_ref[:]
    coef = jnp.dot(alpha, mat_ref[:], preferred_element_type=jnp.float32,
                   precision=lax.Precision.HIGHEST)
    coef_ref[:] = coef + scal_ref[:, 3:4] * msk_ref[:]
    srec_ref[:] = jnp.concatenate(
        [scal_ref[:, 0:3], jnp.zeros((_C, 5), jnp.float32)], axis=1)


def _prep_tables(alpha, scal):
    return pl.pallas_call(
        _prep_body,
        out_shape=[
            jax.ShapeDtypeStruct((_C, 4 * _SPAD), jnp.float32),
            jax.ShapeDtypeStruct((_C, 8), jnp.float32),
        ],
    )(alpha, scal, jnp.asarray(_M_NP), jnp.asarray(_MSK_NP))


def _make_sc_kernel(nslice):
    rows = 224
    cols = 224
    cps = 4
    crows = rows // cps
    nw = 32
    chunks = nslice * cps
    assert chunks % nw == 0
    cpw = chunks // nw
    assert cpw % 2 == 0
    cvregs = cols // 16
    plane = _C * _SPAD

    mesh = plsc.VectorSubcoreMesh(core_axis_name="c", subcore_axis_name="s")

    @functools.partial(
        pl.kernel,
        out_type=jax.ShapeDtypeStruct((nslice, rows, cols), jnp.float32),
        mesh=mesh,
        scratch_types=[
            [pltpu.VMEM((plane,), jnp.float32) for _ in range(4)],
            pltpu.VMEM((_C * 8,), jnp.float32),
            [pltpu.VMEM((crows, cols), jnp.float32) for _ in range(2)],
            [pltpu.VMEM((crows, cols), jnp.float32) for _ in range(2)],
            [pltpu.SemaphoreType.DMA for _ in range(4)],
        ],
        compiler_params=pltpu.CompilerParams(needs_layout_passes=False),
    )
    def sc_kernel(x_hbm, planes_hbm, srec_hbm, out_hbm,
                  tbl_v, srec_v, xin_v, yout_v, sems):
        wid = lax.axis_index("s") * 2 + lax.axis_index("c")
        for d in range(4):
            pltpu.sync_copy(planes_hbm.at[pl.ds(d * plane, plane)], tbl_v[d])
        pltpu.sync_copy(srec_hbm, srec_v)
        g0 = wid * cpw

        def in_copy(k, buf, sem):
            g = g0 + k
            return pltpu.make_async_copy(
                x_hbm.at[g // cps, pl.ds((g % cps) * crows, crows), :],
                buf, sem)

        def out_copy(k, buf, sem):
            g = g0 + k
            return pltpu.make_async_copy(
                buf, out_hbm.at[g // cps, pl.ds((g % cps) * crows, crows), :],
                sem)

        def compute(k, xv, yv):
            c16 = (((g0 + k) // cps) % _C) * _SPAD
            c16f = c16.astype(jnp.float32)
            ccv = jnp.full((16,), c16f + _INVH)
            blov = jnp.full((16,), c16f)
            bhiv = jnp.full((16,), c16f + 12.0)
            sidx = jnp.full((16,), (((g0 + k) // cps) % _C) * 8, jnp.int32)
            av = plsc.load_gather(srec_v, [sidx])
            bv = plsc.load_gather(srec_v, [sidx + 1])
            gv = plsc.load_gather(srec_v, [sidx + 2])

            @plsc.parallel_loop(0, crows, unroll=2)
            def _rloop(r):
                for ci in range(cvregs):
                    xx = xv[r, pl.ds(ci * 16, 16)]
                    xa = jnp.minimum(jnp.maximum(xx * av + bv, -_CLAMP),
                                     _CLAMP)
                    t = jnp.minimum(jnp.maximum(xa * _INVH + ccv, blov), bhiv)
                    s = t.astype(jnp.int32)
                    c0 = plsc.load_gather(tbl_v[0], [s])
                    c1 = plsc.load_gather(tbl_v[1], [s])
                    c2 = plsc.load_gather(tbl_v[2], [s])
                    c3 = plsc.load_gather(tbl_v[3], [s])
                    rr = ((c3 * xa + c2) * xa + c1) * xa + c0
                    yv[r, pl.ds(ci * 16, 16)] = xx * gv + rr

        in_copy(0, xin_v[0], sems[0]).start()

        @pl.loop(0, cpw // 2)
        def _outer(k2):
            k = k2 * 2
            in_copy(k + 1, xin_v[1], sems[1]).start()
            in_copy(k, xin_v[0], sems[0]).wait()

            @pl.when(k2 > 0)
            def _():
                out_copy(k - 2, yout_v[0], sems[2]).wait()

            compute(k, xin_v[0], yout_v[0])
            out_copy(k, yout_v[0], sems[2]).start()

            @pl.when(k + 2 < cpw)
            def _():
                in_copy(k + 2, xin_v[0], sems[0]).start()

            in_copy(k + 1, xin_v[1], sems[1]).wait()

            @pl.when(k2 > 0)
            def _():
                out_copy(k - 1, yout_v[1], sems[3]).wait()

            compute(k + 1, xin_v[1], yout_v[1])
            out_copy(k + 1, yout_v[1], sems[3]).start()

        out_copy(cpw - 2, yout_v[0], sems[2]).wait()
        out_copy(cpw - 1, yout_v[1], sems[3]).wait()

    return sc_kernel


def kernel(x, a, b, alpha, id_gain, bias):
    scal = jnp.stack([a, b, id_gain, bias], axis=1)
    coef, srec = _prep_tables(alpha, scal)
    planes = coef.reshape(_C, 4, _SPAD).transpose(1, 0, 2).reshape(-1)
    B, Cc, H, W = x.shape
    sc = _make_sc_kernel(B * Cc)
    y = sc(x.reshape(B * Cc, H, W), planes, srec.reshape(-1))
    return y.reshape(x.shape)


# speedup vs baseline: 2.2653x; 1.4702x over previous
import functools

import numpy as np
import jax
import jax.numpy as jnp
from jax import lax
from jax.experimental import pallas as pl
from jax.experimental.pallas import tpu as pltpu
from jax.experimental.pallas import tpu_sc as plsc

_C = 192
_K = 16
_P = 3
_CLAMP = 1.5
_NSPAN = _K - _P
_SPAD = 16
_INVH = 6.5


def _knots_f64():
    n_int = _K - _P - 1
    interior = np.linspace(-1.0, 1.0, n_int + 2)[1:-1]
    return np.concatenate(
        [np.full(_P + 1, -1.0), interior, np.full(_P + 1, 1.0)])


def _local_basis_f64(x, i, kn):
    js = np.arange(1, _P + 1)
    left = x - kn[i + 1 - js]
    right = kn[i + js] - x
    N = np.zeros(_P + 1)
    N[0] = 1.0
    for j in range(1, _P + 1):
        saved = 0.0
        for r in range(j):
            denom = right[r] + left[j - r - 1]
            temp = N[r] / denom
            N[r] = saved + right[r] * temp
            saved = left[j - r - 1] * temp
        N[j] = saved
    return N


def _basis_matrices():
    kn = _knots_f64()
    M = np.zeros((_K, 4 * _SPAD))
    for s in range(_NSPAN):
        i = s + _P
        t0, t1 = kn[i], kn[i + 1]
        xs = t0 + (t1 - t0) * np.array([0.1, 0.35, 0.65, 0.9])
        V = np.vander(xs, 4, increasing=True)
        Nm = np.stack([_local_basis_f64(x, i, kn) for x in xs])
        Bj = np.linalg.solve(V, Nm).T
        for j in range(4):
            for d in range(4):
                M[s + j, d * _SPAD + s] = Bj[j, d]
    mask = np.zeros((1, 4 * _SPAD))
    mask[0, :_NSPAN] = 1.0
    return M.astype(np.float32), mask.astype(np.float32)


_M_NP, _MSK_NP = _basis_matrices()


def _prep_body(alpha_ref, scal_ref, mat_ref, msk_ref, coef_ref, srec_ref):
    alpha = alpha_ref[:]
    coef = jnp.dot(alpha, mat_ref[:], preferred_element_type=jnp.float32,
                   precision=lax.Precision.HIGHEST)
    coef_ref[:] = coef + scal_ref[:, 3:4] * msk_ref[:]
    srec_ref[:] = jnp.concatenate(
        [scal_ref[:, 0:3], jnp.zeros((_C, 5), jnp.float32)], axis=1)


def _prep_tables(alpha, scal):
    return pl.pallas_call(
        _prep_body,
        out_shape=[
            jax.ShapeDtypeStruct((_C, 4 * _SPAD), jnp.float32),
            jax.ShapeDtypeStruct((_C, 8), jnp.float32),
        ],
    )(alpha, scal, jnp.asarray(_M_NP), jnp.asarray(_MSK_NP))


def _make_sc_kernel(nslice):
    rows = 224
    cols = 224
    cps = 4
    crows = rows // cps
    nw = 32
    chunks = nslice * cps
    assert chunks % nw == 0
    cpw = chunks // nw
    assert cpw % 2 == 0
    cvregs = cols // 16
    plane = _C * _SPAD

    mesh = plsc.VectorSubcoreMesh(core_axis_name="c", subcore_axis_name="s")

    @functools.partial(
        pl.kernel,
        out_type=jax.ShapeDtypeStruct((nslice, rows, cols), jnp.float32),
        mesh=mesh,
        scratch_types=[
            [pltpu.VMEM((plane,), jnp.float32) for _ in range(4)],
            pltpu.VMEM((_C * 8,), jnp.float32),
            [pltpu.VMEM((crows, cols), jnp.float32) for _ in range(2)],
            [pltpu.VMEM((crows, cols), jnp.float32) for _ in range(2)],
            [pltpu.SemaphoreType.DMA for _ in range(4)],
        ],
        compiler_params=pltpu.CompilerParams(needs_layout_passes=False),
    )
    def sc_kernel(x_hbm, planes_hbm, srec_hbm, out_hbm,
                  tbl_v, srec_v, xin_v, yout_v, sems):
        wid = lax.axis_index("s") * 2 + lax.axis_index("c")
        for d in range(4):
            pltpu.sync_copy(planes_hbm.at[pl.ds(d * plane, plane)], tbl_v[d])
        pltpu.sync_copy(srec_hbm, srec_v)
        g0 = wid * cpw

        def in_copy(k, buf, sem):
            g = g0 + k
            return pltpu.make_async_copy(
                x_hbm.at[g // cps, pl.ds((g % cps) * crows, crows), :],
                buf, sem)

        def out_copy(k, buf, sem):
            g = g0 + k
            return pltpu.make_async_copy(
                buf, out_hbm.at[g // cps, pl.ds((g % cps) * crows, crows), :],
                sem)

        def compute(k, xv, yv):
            c16 = (((g0 + k) // cps) % _C) * _SPAD
            c16f = c16.astype(jnp.float32)
            ccv = jnp.full((16,), c16f + _INVH)
            blov = jnp.full((16,), c16f)
            bhiv = jnp.full((16,), c16f + 12.0)
            sidx = jnp.full((16,), (((g0 + k) // cps) % _C) * 8, jnp.int32)
            av = plsc.load_gather(srec_v, [sidx])
            bv = plsc.load_gather(srec_v, [sidx + 1])
            gv = plsc.load_gather(srec_v, [sidx + 2])

            @plsc.parallel_loop(0, crows, unroll=1)
            def _rloop(r):
                for ci in range(cvregs):
                    xx = xv[r, pl.ds(ci * 16, 16)]
                    xa = jnp.minimum(jnp.maximum(xx * av + bv, -_CLAMP),
                                     _CLAMP)
                    t = jnp.minimum(jnp.maximum(xa * _INVH + ccv, blov), bhiv)
                    s = t.astype(jnp.int32)
                    c0 = plsc.load_gather(tbl_v[0], [s])
                    c1 = plsc.load_gather(tbl_v[1], [s])
                    c2 = plsc.load_gather(tbl_v[2], [s])
                    c3 = plsc.load_gather(tbl_v[3], [s])
                    rr = ((c3 * xa + c2) * xa + c1) * xa + c0
                    yv[r, pl.ds(ci * 16, 16)] = xx * gv + rr

        in_copy(0, xin_v[0], sems[0]).start()

        @pl.loop(0, cpw // 2)
        def _outer(k2):
            k = k2 * 2
            in_copy(k + 1, xin_v[1], sems[1]).start()
            in_copy(k, xin_v[0], sems[0]).wait()

            @pl.when(k2 > 0)
            def _():
                out_copy(k - 2, yout_v[0], sems[2]).wait()

            compute(k, xin_v[0], yout_v[0])
            out_copy(k, yout_v[0], sems[2]).start()

            @pl.when(k + 2 < cpw)
            def _():
                in_copy(k + 2, xin_v[0], sems[0]).start()

            in_copy(k + 1, xin_v[1], sems[1]).wait()

            @pl.when(k2 > 0)
            def _():
                out_copy(k - 1, yout_v[1], sems[3]).wait()

            compute(k + 1, xin_v[1], yout_v[1])
            out_copy(k + 1, yout_v[1], sems[3]).start()

        out_copy(cpw - 2, yout_v[0], sems[2]).wait()
        out_copy(cpw - 1, yout_v[1], sems[3]).wait()

    return sc_kernel


def kernel(x, a, b, alpha, id_gain, bias):
    scal = jnp.stack([a, b, id_gain, bias], axis=1)
    coef, srec = _prep_tables(alpha, scal)
    planes = coef.reshape(_C, 4, _SPAD).transpose(1, 0, 2).reshape(-1)
    B, Cc, H, W = x.shape
    sc = _make_sc_kernel(B * Cc)
    y = sc(x.reshape(B * Cc, H, W), planes, srec.reshape(-1))
    return y.reshape(x.shape)


# speedup vs baseline: 2.3014x; 1.0160x over previous
import functools

import numpy as np
import jax
import jax.numpy as jnp
from jax import lax
from jax.experimental import pallas as pl
from jax.experimental.pallas import tpu as pltpu
from jax.experimental.pallas import tpu_sc as plsc

_C = 192
_K = 16
_P = 3
_CLAMP = 1.5
_NSPAN = _K - _P
_SPAD = 16
_INVH = 6.5


def _knots_f64():
    n_int = _K - _P - 1
    interior = np.linspace(-1.0, 1.0, n_int + 2)[1:-1]
    return np.concatenate(
        [np.full(_P + 1, -1.0), interior, np.full(_P + 1, 1.0)])


def _local_basis_f64(x, i, kn):
    js = np.arange(1, _P + 1)
    left = x - kn[i + 1 - js]
    right = kn[i + js] - x
    N = np.zeros(_P + 1)
    N[0] = 1.0
    for j in range(1, _P + 1):
        saved = 0.0
        for r in range(j):
            denom = right[r] + left[j - r - 1]
            temp = N[r] / denom
            N[r] = saved + right[r] * temp
            saved = left[j - r - 1] * temp
        N[j] = saved
    return N


def _basis_matrices():
    kn = _knots_f64()
    M = np.zeros((_K, 4 * _SPAD))
    for s in range(_NSPAN):
        i = s + _P
        t0, t1 = kn[i], kn[i + 1]
        xs = t0 + (t1 - t0) * np.array([0.1, 0.35, 0.65, 0.9])
        V = np.vander(xs, 4, increasing=True)
        Nm = np.stack([_local_basis_f64(x, i, kn) for x in xs])
        Bj = np.linalg.solve(V, Nm).T
        for j in range(4):
            for d in range(4):
                M[s + j, d * _SPAD + s] = Bj[j, d]
    mask = np.zeros((1, 4 * _SPAD))
    mask[0, :_NSPAN] = 1.0
    return M.astype(np.float32), mask.astype(np.float32)


_M_NP, _MSK_NP = _basis_matrices()


def _prep_body(alpha_ref, scal_ref, mat_ref, msk_ref, coef_ref, srec_ref):
    alpha = alpha_ref[:]
    coef = jnp.dot(alpha, mat_ref[:], preferred_element_type=jnp.float32,
                   precision=lax.Precision.HIGHEST)
    coef_ref[:] = coef + scal_ref[:, 3:4] * msk_ref[:]
    srec_ref[:] = jnp.concatenate(
        [scal_ref[:, 0:3], jnp.zeros((_C, 5), jnp.float32)], axis=1)


def _prep_tables(alpha, scal):
    return pl.pallas_call(
        _prep_body,
        out_shape=[
            jax.ShapeDtypeStruct((_C, 4 * _SPAD), jnp.float32),
            jax.ShapeDtypeStruct((_C, 8), jnp.float32),
        ],
    )(alpha, scal, jnp.asarray(_M_NP), jnp.asarray(_MSK_NP))


def _make_sc_kernel(nslice):
    rows = 224
    cols = 224
    cps = 2
    crows = rows // cps
    nw = 32
    chunks = nslice * cps
    assert chunks % nw == 0
    cpw = chunks // nw
    assert cpw % 2 == 0
    cvregs = cols // 16
    plane = _C * _SPAD

    mesh = plsc.VectorSubcoreMesh(core_axis_name="c", subcore_axis_name="s")

    @functools.partial(
        pl.kernel,
        out_type=jax.ShapeDtypeStruct((nslice, rows, cols), jnp.float32),
        mesh=mesh,
        scratch_types=[
            [pltpu.VMEM((plane,), jnp.float32) for _ in range(4)],
            pltpu.VMEM((_C * 8,), jnp.float32),
            [pltpu.VMEM((crows, cols), jnp.float32) for _ in range(2)],
            [pltpu.VMEM((crows, cols), jnp.float32) for _ in range(2)],
            [pltpu.SemaphoreType.DMA for _ in range(4)],
        ],
        compiler_params=pltpu.CompilerParams(needs_layout_passes=False),
    )
    def sc_kernel(x_hbm, planes_hbm, srec_hbm, out_hbm,
                  tbl_v, srec_v, xin_v, yout_v, sems):
        wid = lax.axis_index("s") * 2 + lax.axis_index("c")
        for d in range(4):
            pltpu.sync_copy(planes_hbm.at[pl.ds(d * plane, plane)], tbl_v[d])
        pltpu.sync_copy(srec_hbm, srec_v)
        g0 = wid * cpw

        def in_copy(k, buf, sem):
            g = g0 + k
            return pltpu.make_async_copy(
                x_hbm.at[g // cps, pl.ds((g % cps) * crows, crows), :],
                buf, sem)

        def out_copy(k, buf, sem):
            g = g0 + k
            return pltpu.make_async_copy(
                buf, out_hbm.at[g // cps, pl.ds((g % cps) * crows, crows), :],
                sem)

        def compute(k, xv, yv):
            c16 = (((g0 + k) // cps) % _C) * _SPAD
            c16f = c16.astype(jnp.float32)
            ccv = jnp.full((16,), c16f + _INVH)
            blov = jnp.full((16,), c16f)
            bhiv = jnp.full((16,), c16f + 12.0)
            sidx = jnp.full((16,), (((g0 + k) // cps) % _C) * 8, jnp.int32)
            av = plsc.load_gather(srec_v, [sidx])
            bv = plsc.load_gather(srec_v, [sidx + 1])
            gv = plsc.load_gather(srec_v, [sidx + 2])

            @plsc.parallel_loop(0, crows, unroll=1)
            def _rloop(r):
                for ci in range(cvregs):
                    xx = xv[r, pl.ds(ci * 16, 16)]
                    xa = jnp.minimum(jnp.maximum(xx * av + bv, -_CLAMP),
                                     _CLAMP)
                    t = jnp.minimum(jnp.maximum(xa * _INVH + ccv, blov), bhiv)
                    s = t.astype(jnp.int32)
                    c0 = plsc.load_gather(tbl_v[0], [s])
                    c1 = plsc.load_gather(tbl_v[1], [s])
                    c2 = plsc.load_gather(tbl_v[2], [s])
                    c3 = plsc.load_gather(tbl_v[3], [s])
                    rr = ((c3 * xa + c2) * xa + c1) * xa + c0
                    yv[r, pl.ds(ci * 16, 16)] = xx * gv + rr

        in_copy(0, xin_v[0], sems[0]).start()

        @pl.loop(0, cpw // 2)
        def _outer(k2):
            k = k2 * 2
            in_copy(k + 1, xin_v[1], sems[1]).start()
            in_copy(k, xin_v[0], sems[0]).wait()

            @pl.when(k2 > 0)
            def _():
                out_copy(k - 2, yout_v[0], sems[2]).wait()

            compute(k, xin_v[0], yout_v[0])
            out_copy(k, yout_v[0], sems[2]).start()

            @pl.when(k + 2 < cpw)
            def _():
                in_copy(k + 2, xin_v[0], sems[0]).start()

            in_copy(k + 1, xin_v[1], sems[1]).wait()

            @pl.when(k2 > 0)
            def _():
                out_copy(k - 1, yout_v[1], sems[3]).wait()

            compute(k + 1, xin_v[1], yout_v[1])
            out_copy(k + 1, yout_v[1], sems[3]).start()

        out_copy(cpw - 2, yout_v[0], sems[2]).wait()
        out_copy(cpw - 1, yout_v[1], sems[3]).wait()

    return sc_kernel


def kernel(x, a, b, alpha, id_gain, bias):
    scal = jnp.stack([a, b, id_gain, bias], axis=1)
    coef, srec = _prep_tables(alpha, scal)
    planes = coef.reshape(_C, 4, _SPAD).transpose(1, 0, 2).reshape(-1)
    B, Cc, H, W = x.shape
    sc = _make_sc_kernel(B * Cc)
    y = sc(x.reshape(B * Cc, H, W), planes, srec.reshape(-1))
    return y.reshape(x.shape)
